# baseline (device time: 181240 ns/iter reference)
import jax
import jax.numpy as jnp
from jax import lax
from jax.experimental import pallas as pl
from jax.experimental.pallas import tpu as pltpu

N_DEV = 16
N_STREAMS = 8
S = 3


def kernel(x, w_mat, scale_x, scale_w):
    m_glob, k_loc = x.shape
    _, n = w_mat.shape
    m_chunk = m_glob // N_DEV
    n_half = n // 2
    n_sub = n // N_STREAMS

    def body(x_ref, w_ref, sx_ref, sw_ref, out_ref,
             buf_ref, w_bf16_ref, tmp_ref, send_sems, recv_sems, credit_sems):
        my = lax.axis_index("i")
        left = lax.rem(my - 1 + N_DEV, N_DEV)
        right = lax.rem(my + 1, N_DEV)

        half_s = N_STREAMS // 2
        peer_out = [right] * half_s + [left] * half_s
        peer_in = [left] * half_s + [right] * half_s

        barrier = pltpu.get_barrier_semaphore()
        for nbr in (left, right):
            pl.semaphore_signal(
                barrier, inc=1,
                device_id=(nbr,), device_id_type=pl.DeviceIdType.MESH,
            )
        pl.semaphore_wait(barrier, 2)

        w_bf16_ref[...] = w_ref[...].astype(jnp.bfloat16)

        def dot_chunk(idx, col0, ncol):
            xa = x_ref[pl.ds(idx * m_chunk, m_chunk), :].astype(jnp.bfloat16)
            return lax.dot_general(
                xa, w_bf16_ref[:, col0:col0 + ncol], (((1,), (0,)), ((), ())),
                preferred_element_type=jnp.float32,
            )

        def mod(v):
            return lax.rem(v + 2 * N_DEV, N_DEV)

        init_idx = [mod(my - 1)] * half_s + [mod(my + 1)] * half_s

        order = tuple(
            k + half_s * p for k in range(half_s) for p in range(2)
        )
        alpha = sx_ref[0] * sw_ref[0]

        def make_rdma(k, h):
            return pltpu.make_async_remote_copy(
                src_ref=buf_ref.at[k, h % S],
                dst_ref=buf_ref.at[k, (h + 1) % S],
                send_sem=send_sems.at[k, h % S],
                recv_sem=recv_sems.at[k, (h + 1) % S],
                device_id=(peer_out[k],),
                device_id_type=pl.DeviceIdType.MESH,
            )

        rdmas = [make_rdma(k, 0) for k in range(N_STREAMS)]
        for k in order:
            buf_ref[k, 0] = dot_chunk(init_idx[k], k * n_sub, n_sub)
            rdmas[k].start()

        for h in range(N_DEV - 1):
            r = (h + 1) % S

            idx_r = mod(my - 2 - h)
            idx_l = mod(my + 2 + h)
            tmp_ref[:, :n_half] = dot_chunk(idx_r, 0, n_half)
            tmp_ref[:, n_half:] = dot_chunk(idx_l, n_half, n_half)

            prev = list(rdmas)
            for k in order:
                cols = pl.ds(k * n_sub, n_sub)
                prev[k].wait_recv()
                if h < N_DEV - 2:
                    buf_ref[k, r] = buf_ref[k, r] + tmp_ref[:, cols]
                    if h + 1 >= S - 1:
                        pl.semaphore_wait(credit_sems.at[k], 1)
                    rdmas[k] = make_rdma(k, h + 1)
                    rdmas[k].start()
                else:
                    y = (buf_ref[k, r] + tmp_ref[:, cols]) * alpha
                    yc = jnp.clip(y, -60.0, 60.0)
                    out_ref[:, cols] = y / (1.0 + jnp.exp(-yc))
            for k in order:
                prev[k].wait_send()
                if h < N_DEV - S:
                    pl.semaphore_signal(
                        credit_sems.at[k], inc=1,
                        device_id=(peer_in[k],),
                        device_id_type=pl.DeviceIdType.MESH,
                    )

    return pl.pallas_call(
        body,
        out_shape=jax.ShapeDtypeStruct((m_chunk, n), jnp.float32),
        in_specs=[
            pl.BlockSpec(memory_space=pltpu.VMEM),
            pl.BlockSpec(memory_space=pltpu.VMEM),
            pl.BlockSpec(memory_space=pltpu.SMEM),
            pl.BlockSpec(memory_space=pltpu.SMEM),
        ],
        out_specs=pl.BlockSpec(memory_space=pltpu.VMEM),
        scratch_shapes=[
            pltpu.VMEM((N_STREAMS, S, m_chunk, n_sub), jnp.float32),
            pltpu.VMEM((k_loc, n), jnp.bfloat16),
            pltpu.VMEM((m_chunk, n), jnp.float32),
            pltpu.SemaphoreType.DMA((N_STREAMS, S)),
            pltpu.SemaphoreType.DMA((N_STREAMS, S)),
            pltpu.SemaphoreType.REGULAR((N_STREAMS,)),
        ],
        compiler_params=pltpu.CompilerParams(collective_id=0),
    )(x, w_mat, scale_x, scale_w)


# device time: 96687 ns/iter; 1.8745x vs baseline; 1.8745x over previous
import jax
import jax.numpy as jnp
from jax import lax
from jax.experimental import pallas as pl
from jax.experimental.pallas import tpu as pltpu

N_DEV = 16
N_STREAMS = 4
S = 3


def kernel(x, w_mat, scale_x, scale_w):
    m_glob, k_loc = x.shape
    _, n = w_mat.shape
    m_chunk = m_glob // N_DEV
    n_half = n // 2
    n_sub = n // N_STREAMS

    def body(x_ref, w_ref, sx_ref, sw_ref, out_ref,
             buf_ref, w_bf16_ref, tmp_ref, send_sems, recv_sems, credit_sems):
        my = lax.axis_index("i")
        left = lax.rem(my - 1 + N_DEV, N_DEV)
        right = lax.rem(my + 1, N_DEV)

        peer_out = [right, right, left, left]
        peer_in = [left, left, right, right]

        barrier = pltpu.get_barrier_semaphore()
        for nbr in (left, right):
            pl.semaphore_signal(
                barrier, inc=1,
                device_id=(nbr,), device_id_type=pl.DeviceIdType.MESH,
            )
        pl.semaphore_wait(barrier, 2)

        w_bf16_ref[...] = w_ref[...].astype(jnp.bfloat16)

        def dot_chunk(idx, col0, ncol):
            xa = x_ref[pl.ds(idx * m_chunk, m_chunk), :].astype(jnp.bfloat16)
            return lax.dot_general(
                xa, w_bf16_ref[:, col0:col0 + ncol], (((1,), (0,)), ((), ())),
                preferred_element_type=jnp.float32,
            )

        def mod(v):
            return lax.rem(v + 2 * N_DEV, N_DEV)

        init_idx = [mod(my - 1), mod(my - 1), mod(my + 1), mod(my + 1)]

        order = (0, 2, 1, 3)
        alpha = sx_ref[0] * sw_ref[0]

        def make_rdma(k, h):
            return pltpu.make_async_remote_copy(
                src_ref=buf_ref.at[k, h % S],
                dst_ref=buf_ref.at[k, (h + 1) % S],
                send_sem=send_sems.at[k, h % S],
                recv_sem=recv_sems.at[k, (h + 1) % S],
                device_id=(peer_out[k],),
                device_id_type=pl.DeviceIdType.MESH,
            )

        rdmas = [make_rdma(k, 0) for k in range(N_STREAMS)]
        for k in order:
            buf_ref[k, 0] = dot_chunk(
                init_idx[k], k * n_sub, n_sub
            ).astype(jnp.bfloat16)
            rdmas[k].start()

        for h in range(N_DEV - 1):
            r = (h + 1) % S

            idx_r = mod(my - 2 - h)
            idx_l = mod(my + 2 + h)
            tmp_ref[:, :n_half] = dot_chunk(idx_r, 0, n_half)
            tmp_ref[:, n_half:] = dot_chunk(idx_l, n_half, n_half)

            prev = list(rdmas)
            for k in order:
                cols = pl.ds(k * n_sub, n_sub)
                prev[k].wait_recv()
                if h < N_DEV - 2:
                    buf_ref[k, r] = (
                        buf_ref[k, r].astype(jnp.float32) + tmp_ref[:, cols]
                    ).astype(jnp.bfloat16)
                    if h + 1 >= S - 1:
                        pl.semaphore_wait(credit_sems.at[k], 1)
                    rdmas[k] = make_rdma(k, h + 1)
                    rdmas[k].start()
                else:
                    y = (
                        buf_ref[k, r].astype(jnp.float32) + tmp_ref[:, cols]
                    ) * alpha
                    yc = jnp.clip(y, -60.0, 60.0)
                    out_ref[:, cols] = y / (1.0 + jnp.exp(-yc))
            for k in order:
                prev[k].wait_send()
                if h < N_DEV - S:
                    pl.semaphore_signal(
                        credit_sems.at[k], inc=1,
                        device_id=(peer_in[k],),
                        device_id_type=pl.DeviceIdType.MESH,
                    )

    return pl.pallas_call(
        body,
        out_shape=jax.ShapeDtypeStruct((m_chunk, n), jnp.float32),
        in_specs=[
            pl.BlockSpec(memory_space=pltpu.VMEM),
            pl.BlockSpec(memory_space=pltpu.VMEM),
            pl.BlockSpec(memory_space=pltpu.SMEM),
            pl.BlockSpec(memory_space=pltpu.SMEM),
        ],
        out_specs=pl.BlockSpec(memory_space=pltpu.VMEM),
        scratch_shapes=[
            pltpu.VMEM((N_STREAMS, S, m_chunk, n_sub), jnp.bfloat16),
            pltpu.VMEM((k_loc, n), jnp.bfloat16),
            pltpu.VMEM((m_chunk, n), jnp.float32),
            pltpu.SemaphoreType.DMA((N_STREAMS, S)),
            pltpu.SemaphoreType.DMA((N_STREAMS, S)),
            pltpu.SemaphoreType.REGULAR((N_STREAMS,)),
        ],
        compiler_params=pltpu.CompilerParams(collective_id=0),
    )(x, w_mat, scale_x, scale_w)
